# baseline (speedup 1.0000x reference)
import functools

import jax
import jax.numpy as jnp
from jax import lax
from jax.experimental import pallas as pl
from jax.experimental.pallas import tpu as pltpu
from jax.experimental.pallas import tpu_sc as plsc

B, C, N = 2, 64, 131072
H, W = 512, 512
HW = H * W
BN = B * N
BHW = B * HW

NC, NS = 2, 16
NW = NC * NS
CHUNK = 1024
NCHUNK = BHW // CHUNK
ROUNDS = NCHUNK // NW
LIST_CAP = 16384
CLIST_CAP = 1024
GB = 64
SEG_WIN = 4096
NWIN = BN // SEG_WIN

_NEG_INF = float("-inf")


def _seg_body(ix_ref, iy_ref, seg_ref):
    i = pl.program_id(0)
    nb = ix_ref.shape[0]
    b_off = jnp.where(i * nb >= N, HW, 0)
    vx = jnp.clip(ix_ref[...], 0, H - 1)
    vy = jnp.clip(iy_ref[...], 0, W - 1)
    seg_ref[...] = b_off + vx * W + vy


def _seg_call(ix, iy):
    nb = 32768
    return pl.pallas_call(
        _seg_body,
        grid=(BN // nb,),
        in_specs=[pl.BlockSpec((nb,), lambda i: (i,)),
                  pl.BlockSpec((nb,), lambda i: (i,))],
        out_specs=pl.BlockSpec((nb,), lambda i: (i,)),
        out_shape=jax.ShapeDtypeStruct((BN,), jnp.int32),
    )(ix, iy)


def _pro_body(fa_ref, fb_ref, fp_ref):
    fp_ref[:, 0:C] = fa_ref[0].T
    fp_ref[:, C:2 * C] = fb_ref[0].T


def _prologue(feat):
    nb = 8192
    nblk = N // nb
    return pl.pallas_call(
        _pro_body,
        grid=(nblk,),
        in_specs=[
            pl.BlockSpec((1, C, nb), lambda i: (0, 0, i)),
            pl.BlockSpec((1, C, nb), lambda i: (1, 0, i)),
        ],
        out_specs=pl.BlockSpec((nb, 2 * C), lambda i: (i, 0)),
        out_shape=jax.ShapeDtypeStruct((N, 2 * C), jnp.float32),
    )(feat, feat)


def _sc_body(feats_hbm, seg_hbm, temp_hbm,
             seg_buf, plist, cpid, clv, rows, accum, sem, sem_w):
    wid = lax.axis_index("s") * NC + lax.axis_index("c")
    lanes = lax.iota(jnp.int32, 16)

    def _pf(i, _):
        cpid[pl.ds(i * 16, 16)] = jnp.zeros((16,), jnp.int32)
        return 0
    lax.fori_loop(0, CLIST_CAP // 16, _pf, 0)

    pltpu.async_copy(seg_hbm.at[pl.ds(0, SEG_WIN)], seg_buf.at[0], sem)

    def _win(w, off):
        p = w & 1
        pltpu.make_async_copy(seg_hbm.at[pl.ds(w * SEG_WIN, SEG_WIN)],
                              seg_buf.at[p], sem).wait()

        @pl.when(w + 1 < NWIN)
        def _():
            pltpu.async_copy(
                seg_hbm.at[pl.ds((w + 1) * SEG_WIN, SEG_WIN)],
                seg_buf.at[(w + 1) & 1], sem)

        def _vreg4(q, off):
            svs, cnts = [], []
            for t in range(4):
                s = seg_buf[p, pl.ds(q * 64 + t * 16, 16)]
                own = ((s >> 10) & (NW - 1)) == wid
                cnts.append(plsc.all_reduce_population_count(own)[0])
                r = (s >> 15) & (ROUNDS - 1)
                lv = s & (CHUNK - 1)
                pid = w * SEG_WIN + q * 64 + t * 16 + lanes
                packed = (pid.astype(jnp.uint32) << 14) | \
                         ((r << 10) | lv).astype(jnp.uint32)
                key = jnp.where(own, jnp.uint32(0), jnp.uint32(1))
                _, sv = plsc.sort_key_val(key, packed)
                svs.append(sv)
            for t in range(4):
                o = jnp.minimum(off, LIST_CAP - 16)
                plist[pl.ds(o, 16)] = svs[t]
                off = off + cnts[t]
            return off
        return lax.fori_loop(0, SEG_WIN // 64, _vreg4, off)

    m_total = jnp.minimum(lax.fori_loop(0, NWIN, _win, jnp.int32(0)),
                          LIST_CAP)
    n_mv4 = (m_total + 63) // 64

    def _round(r, _):
        pair_base = (r * NW + wid) * (CHUNK // 2)

        def _scan4(q, k):
            svs, cnts = [], []
            for t in range(4):
                i0 = q * 64 + t * 16
                pk = plist[pl.ds(i0, 16)]
                valid = (i0 + lanes) < m_total
                r_of = ((pk >> 10) &
                        jnp.uint32(ROUNDS - 1)).astype(jnp.int32)
                sel = jnp.logical_and(r_of == r, valid)
                cnts.append(plsc.all_reduce_population_count(sel)[0])
                key = jnp.where(sel, jnp.uint32(0), jnp.uint32(1))
                _, sv = plsc.sort_key_val(key, pk)
                svs.append(sv)
            for t in range(4):
                o = jnp.minimum(k, CLIST_CAP - 16)
                cpid[pl.ds(o, 16)] = ((svs[t] >> 14) &
                                      jnp.uint32(N - 1)).astype(jnp.int32)
                clv[pl.ds(o, 16)] = ((svs[t] & jnp.uint32(CHUNK - 1)) |
                                     ((svs[t] >> 31) << 11)).astype(jnp.int32)
                k = k + cnts[t]
            return k
        k_total = jnp.minimum(lax.fori_loop(0, n_mv4, _scan4, jnp.int32(0)),
                              CLIST_CAP)
        clv[pl.ds(k_total, 16)] = jnp.full((16,), CHUNK, jnp.int32)

        @pl.when(r > 0)
        def _():
            pltpu.make_async_copy(
                accum.at[pl.ds(0, CHUNK // 2), :],
                temp_hbm.at[pl.ds(pair_base, CHUNK // 2), :], sem_w).wait()

        def _init(i, _):
            for u in range(4):
                for cg in range(8):
                    accum[i * 4 + u, pl.ds(cg * 16, 16)] = jnp.full(
                        (16,), _NEG_INF, jnp.float32)
            return 0
        lax.fori_loop(0, CHUNK // 2 // 4, _init, 0)

        nb = (k_total + GB - 1) // GB

        @pl.when(nb > 0)
        def _():
            pltpu.async_copy(
                feats_hbm.at[cpid.at[pl.ds(0, GB)]], rows.at[0], sem)

        def _batch(g, _):
            pb = g & 1
            pltpu.make_async_copy(
                feats_hbm.at[cpid.at[pl.ds(g * GB, GB)]],
                rows.at[pb], sem).wait()

            @pl.when(g + 1 < nb)
            def _():
                pltpu.async_copy(
                    feats_hbm.at[cpid.at[pl.ds((g + 1) * GB, GB)]],
                    rows.at[(g + 1) & 1], sem)
            cnt = jnp.minimum(k_total - g * GB, GB)

            def _grp(j, _):
                lvv = clv[pl.ds(g * GB + j * 16, 16)]
                for u in range(16):
                    e = lvv[u]
                    lv = e & 2047
                    hoff = ((e >> 11) & 1) << 6
                    trash = lv > CHUNK - 1
                    row = jnp.where(trash, CHUNK // 2, lv & (CHUNK // 2 - 1))
                    coff = jnp.where(trash, 0, (lv >> 9) << 6)
                    for cg in range(4):
                        a = accum[row, pl.ds(coff + cg * 16, 16)]
                        f = rows[pb, j * 16 + u, pl.ds(hoff + cg * 16, 16)]
                        accum[row, pl.ds(coff + cg * 16, 16)] = \
                            jnp.maximum(a, f)
                return 0
            return lax.fori_loop(0, (cnt + 15) // 16, _grp, 0)
        lax.fori_loop(0, nb, _batch, 0)

        pltpu.async_copy(accum.at[pl.ds(0, CHUNK // 2), :],
                         temp_hbm.at[pl.ds(pair_base, CHUNK // 2), :], sem_w)
        return 0
    lax.fori_loop(0, ROUNDS, _round, 0)
    pltpu.make_async_copy(
        accum.at[pl.ds(0, CHUNK // 2), :],
        temp_hbm.at[pl.ds((ROUNDS - 1) * NW * (CHUNK // 2), CHUNK // 2), :],
        sem_w).wait()


@functools.partial(
    pl.kernel,
    out_type=jax.ShapeDtypeStruct((BHW // 2, 2 * C), jnp.float32),
    mesh=plsc.VectorSubcoreMesh(core_axis_name="c", subcore_axis_name="s",
                                num_cores=NC, num_subcores=NS),
    scratch_types=[
        pltpu.VMEM((2, SEG_WIN), jnp.int32),
        pltpu.VMEM((LIST_CAP + 64,), jnp.uint32),
        pltpu.VMEM((CLIST_CAP,), jnp.int32),
        pltpu.VMEM((CLIST_CAP + 16,), jnp.int32),
        pltpu.VMEM((2, GB, 2 * C), jnp.float32),
        pltpu.VMEM((CHUNK // 2 + 1, 2 * C), jnp.float32),
        pltpu.SemaphoreType.DMA,
        pltpu.SemaphoreType.DMA,
    ],
    compiler_params=pltpu.CompilerParams(needs_layout_passes=False),
)
def _sc_kernel(feats_hbm, seg_hbm, temp_hbm,
               seg_buf, plist, cpid, clv, rows, accum, sem, sem_w):
    _sc_body(feats_hbm, seg_hbm, temp_hbm,
             seg_buf, plist, cpid, clv, rows, accum, sem, sem_w)


def _epi_body(temp_ref, out_ref):
    hc = CHUNK // 2
    for q in range(4):
        t = temp_ref[pl.ds(q * hc, hc), :]
        lo = t[:, 0:C]
        hi = t[:, C:2 * C]
        out_ref[0, :, pl.ds(q * CHUNK, hc)] = \
            jnp.where(jnp.isfinite(lo), lo, 0.0).T
        out_ref[0, :, pl.ds(q * CHUNK + hc, hc)] = \
            jnp.where(jnp.isfinite(hi), hi, 0.0).T


def _epilogue(temp):
    nck = 4
    per_b = NCHUNK // B // nck
    out = pl.pallas_call(
        _epi_body,
        grid=(NCHUNK // nck,),
        in_specs=[pl.BlockSpec((nck * CHUNK // 2, 2 * C),
                               lambda i: (i, 0))],
        out_specs=pl.BlockSpec((1, C, nck * CHUNK),
                               lambda i: (i // per_b, 0, i % per_b)),
        out_shape=jax.ShapeDtypeStruct((B, C, HW), jnp.float32),
    )(temp)
    return out.reshape(B, C, H, W)


def kernel(pcds_feat, pcds_ind):
    ix = pcds_ind[:, :, 0, 0].reshape(BN)
    iy = pcds_ind[:, :, 1, 0].reshape(BN)
    seg = _seg_call(ix, iy)
    feats_p = _prologue(pcds_feat[..., 0])
    temp = _sc_kernel(feats_p, seg)
    return _epilogue(temp)


# speedup vs baseline: 1.1402x; 1.1402x over previous
import functools

import jax
import jax.numpy as jnp
from jax import lax
from jax.experimental import pallas as pl
from jax.experimental.pallas import tpu as pltpu
from jax.experimental.pallas import tpu_sc as plsc

B, C, N = 2, 64, 131072
H, W = 512, 512
HW = H * W
BN = B * N
BHW = B * HW

NC, NS = 2, 16
NW = NC * NS
CHUNK = 1024
NCHUNK = BHW // CHUNK
ROUNDS = NCHUNK // NW
LIST_CAP = 16384
CLIST_CAP = 1024
GB = 64
SEG_WIN = 4096
NWIN = BN // SEG_WIN

_NEG_INF = float("-inf")


def _seg_body(ix_ref, iy_ref, seg_ref):
    i = pl.program_id(0)
    nb = ix_ref.shape[0]
    b_off = jnp.where(i * nb >= N, HW, 0)
    vx = jnp.clip(ix_ref[...], 0, H - 1)
    vy = jnp.clip(iy_ref[...], 0, W - 1)
    seg_ref[...] = b_off + vx * W + vy


def _seg_call(ix, iy):
    nb = 32768
    return pl.pallas_call(
        _seg_body,
        grid=(BN // nb,),
        in_specs=[pl.BlockSpec((nb,), lambda i: (i,)),
                  pl.BlockSpec((nb,), lambda i: (i,))],
        out_specs=pl.BlockSpec((nb,), lambda i: (i,)),
        out_shape=jax.ShapeDtypeStruct((BN,), jnp.int32),
    )(ix, iy)


def _pro_body(fa_ref, fb_ref, fp_ref):
    fp_ref[:, 0:C] = fa_ref[0].T
    fp_ref[:, C:2 * C] = fb_ref[0].T


def _prologue(feat):
    nb = 8192
    nblk = N // nb
    return pl.pallas_call(
        _pro_body,
        grid=(nblk,),
        in_specs=[
            pl.BlockSpec((1, C, nb), lambda i: (0, 0, i)),
            pl.BlockSpec((1, C, nb), lambda i: (1, 0, i)),
        ],
        out_specs=pl.BlockSpec((nb, 2 * C), lambda i: (i, 0)),
        out_shape=jax.ShapeDtypeStruct((N, 2 * C), jnp.float32),
    )(feat, feat)


def _sc_body(feats_hbm, seg_hbm, temp_hbm,
             seg_buf, plist, cpid, clv, rows, accum, sem, sem_w):
    wid = lax.axis_index("s") * NC + lax.axis_index("c")
    lanes = lax.iota(jnp.int32, 16)

    def _pf(i, _):
        cpid[pl.ds(i * 16, 16)] = jnp.zeros((16,), jnp.int32)
        return 0
    lax.fori_loop(0, CLIST_CAP // 16, _pf, 0)

    pltpu.async_copy(seg_hbm.at[pl.ds(0, SEG_WIN)], seg_buf.at[0], sem)

    def _win(w, off):
        p = w & 1
        pltpu.make_async_copy(seg_hbm.at[pl.ds(w * SEG_WIN, SEG_WIN)],
                              seg_buf.at[p], sem).wait()

        @pl.when(w + 1 < NWIN)
        def _():
            pltpu.async_copy(
                seg_hbm.at[pl.ds((w + 1) * SEG_WIN, SEG_WIN)],
                seg_buf.at[(w + 1) & 1], sem)

        def _vreg8(q, off):
            svs, cnts = [], []
            for t in range(8):
                s = seg_buf[p, pl.ds(q * 128 + t * 16, 16)]
                own = ((s >> 10) & (NW - 1)) == wid
                cnts.append(plsc.all_reduce_population_count(own)[0])
                r = (s >> 15) & (ROUNDS - 1)
                lv = s & (CHUNK - 1)
                pid = w * SEG_WIN + q * 128 + t * 16 + lanes
                packed = (pid.astype(jnp.uint32) << 14) | \
                         ((r << 10) | lv).astype(jnp.uint32)
                key = jnp.where(own, jnp.uint32(0), jnp.uint32(1))
                _, sv = plsc.sort_key_val(key, packed)
                svs.append(sv)
            for t in range(8):
                o = jnp.minimum(off, LIST_CAP - 16)
                plist[pl.ds(o, 16)] = svs[t]
                off = off + cnts[t]
            return off
        return lax.fori_loop(0, SEG_WIN // 128, _vreg8, off)

    m_total = jnp.minimum(lax.fori_loop(0, NWIN, _win, jnp.int32(0)),
                          LIST_CAP)
    n_mv4 = (m_total + 63) // 64

    def _round(r, _):
        pair_base = (r * NW + wid) * (CHUNK // 2)

        def _scan4(q, k):
            svs, cnts = [], []
            for t in range(4):
                i0 = q * 64 + t * 16
                pk = plist[pl.ds(i0, 16)]
                valid = (i0 + lanes) < m_total
                r_of = ((pk >> 10) &
                        jnp.uint32(ROUNDS - 1)).astype(jnp.int32)
                sel = jnp.logical_and(r_of == r, valid)
                cnts.append(plsc.all_reduce_population_count(sel)[0])
                key = jnp.where(sel, jnp.uint32(0), jnp.uint32(1))
                _, sv = plsc.sort_key_val(key, pk)
                svs.append(sv)
            for t in range(4):
                o = jnp.minimum(k, CLIST_CAP - 16)
                cpid[pl.ds(o, 16)] = ((svs[t] >> 14) &
                                      jnp.uint32(N - 1)).astype(jnp.int32)
                clv[pl.ds(o, 16)] = ((svs[t] & jnp.uint32(CHUNK - 1)) |
                                     ((svs[t] >> 31) << 11)).astype(jnp.int32)
                k = k + cnts[t]
            return k
        k_total = jnp.minimum(lax.fori_loop(0, n_mv4, _scan4, jnp.int32(0)),
                              CLIST_CAP)
        clv[pl.ds(k_total, 16)] = jnp.full((16,), CHUNK, jnp.int32)

        @pl.when(r > 0)
        def _():
            pltpu.make_async_copy(
                accum.at[pl.ds(0, CHUNK // 2), :],
                temp_hbm.at[pl.ds(pair_base, CHUNK // 2), :], sem_w).wait()

        def _init(i, _):
            for u in range(4):
                for cg in range(8):
                    accum[i * 4 + u, pl.ds(cg * 16, 16)] = jnp.full(
                        (16,), _NEG_INF, jnp.float32)
            return 0
        lax.fori_loop(0, CHUNK // 2 // 4, _init, 0)

        nb = (k_total + GB - 1) // GB

        @pl.when(nb > 0)
        def _():
            pltpu.async_copy(
                feats_hbm.at[cpid.at[pl.ds(0, GB)]], rows.at[0], sem)

        def _batch(g, _):
            pb = g & 1
            pltpu.make_async_copy(
                feats_hbm.at[cpid.at[pl.ds(g * GB, GB)]],
                rows.at[pb], sem).wait()

            @pl.when(g + 1 < nb)
            def _():
                pltpu.async_copy(
                    feats_hbm.at[cpid.at[pl.ds((g + 1) * GB, GB)]],
                    rows.at[(g + 1) & 1], sem)
            cnt = jnp.minimum(k_total - g * GB, GB)

            def _grp(j, _):
                lvv = clv[pl.ds(g * GB + j * 16, 16)]
                for u in range(16):
                    e = lvv[u]
                    lv = e & 2047
                    hoff = ((e >> 11) & 1) << 6
                    trash = lv > CHUNK - 1
                    row = jnp.where(trash, CHUNK // 2, lv & (CHUNK // 2 - 1))
                    coff = jnp.where(trash, 0, (lv >> 9) << 6)
                    for cg in range(4):
                        a = accum[row, pl.ds(coff + cg * 16, 16)]
                        f = rows[pb, j * 16 + u, pl.ds(hoff + cg * 16, 16)]
                        accum[row, pl.ds(coff + cg * 16, 16)] = \
                            jnp.maximum(a, f)
                return 0
            return lax.fori_loop(0, (cnt + 15) // 16, _grp, 0)
        lax.fori_loop(0, nb, _batch, 0)

        pltpu.async_copy(accum.at[pl.ds(0, CHUNK // 2), :],
                         temp_hbm.at[pl.ds(pair_base, CHUNK // 2), :], sem_w)
        return 0
    lax.fori_loop(0, ROUNDS, _round, 0)
    pltpu.make_async_copy(
        accum.at[pl.ds(0, CHUNK // 2), :],
        temp_hbm.at[pl.ds((ROUNDS - 1) * NW * (CHUNK // 2), CHUNK // 2), :],
        sem_w).wait()


@functools.partial(
    pl.kernel,
    out_type=jax.ShapeDtypeStruct((BHW // 2, 2 * C), jnp.float32),
    mesh=plsc.VectorSubcoreMesh(core_axis_name="c", subcore_axis_name="s",
                                num_cores=NC, num_subcores=NS),
    scratch_types=[
        pltpu.VMEM((2, SEG_WIN), jnp.int32),
        pltpu.VMEM((LIST_CAP + 64,), jnp.uint32),
        pltpu.VMEM((CLIST_CAP,), jnp.int32),
        pltpu.VMEM((CLIST_CAP + 16,), jnp.int32),
        pltpu.VMEM((2, GB, 2 * C), jnp.float32),
        pltpu.VMEM((CHUNK // 2 + 1, 2 * C), jnp.float32),
        pltpu.SemaphoreType.DMA,
        pltpu.SemaphoreType.DMA,
    ],
    compiler_params=pltpu.CompilerParams(needs_layout_passes=False),
)
def _sc_kernel(feats_hbm, seg_hbm, temp_hbm,
               seg_buf, plist, cpid, clv, rows, accum, sem, sem_w):
    _sc_body(feats_hbm, seg_hbm, temp_hbm,
             seg_buf, plist, cpid, clv, rows, accum, sem, sem_w)


def _epi_body(temp_ref, out_ref):
    hc = CHUNK // 2
    for q in range(4):
        t = temp_ref[pl.ds(q * hc, hc), :]
        lo = t[:, 0:C]
        hi = t[:, C:2 * C]
        out_ref[0, :, 2 * q, :] = jnp.where(jnp.isfinite(lo), lo, 0.0).T
        out_ref[0, :, 2 * q + 1, :] = jnp.where(jnp.isfinite(hi), hi, 0.0).T


def _epilogue(temp):
    nck = 4
    per_b = NCHUNK // B // nck
    return pl.pallas_call(
        _epi_body,
        grid=(NCHUNK // nck,),
        in_specs=[pl.BlockSpec((nck * CHUNK // 2, 2 * C),
                               lambda i: (i, 0))],
        out_specs=pl.BlockSpec((1, C, 2 * nck, W),
                               lambda i: (i // per_b, 0, i % per_b, 0)),
        out_shape=jax.ShapeDtypeStruct((B, C, H, W), jnp.float32),
    )(temp)


def kernel(pcds_feat, pcds_ind):
    ix = pcds_ind[:, :, 0, 0].reshape(BN)
    iy = pcds_ind[:, :, 1, 0].reshape(BN)
    seg = _seg_call(ix, iy)
    feats_p = _prologue(pcds_feat[..., 0])
    temp = _sc_kernel(feats_p, seg)
    return _epilogue(temp)


# speedup vs baseline: 1.1454x; 1.0045x over previous
import functools

import jax
import jax.numpy as jnp
from jax import lax
from jax.experimental import pallas as pl
from jax.experimental.pallas import tpu as pltpu
from jax.experimental.pallas import tpu_sc as plsc

B, C, N = 2, 64, 131072
H, W = 512, 512
HW = H * W
BN = B * N
BHW = B * HW

NC, NS = 2, 16
NW = NC * NS
CHUNK = 1024
NCHUNK = BHW // CHUNK
ROUNDS = NCHUNK // NW
LIST_CAP = 16384
CLIST_CAP = 1024
GB = 64
SEG_WIN = 8192
NWIN = BN // SEG_WIN

_NEG_INF = float("-inf")


def _seg_body(ix_ref, iy_ref, seg_ref):
    i = pl.program_id(0)
    nb = ix_ref.shape[0]
    b_off = jnp.where(i * nb >= N, HW, 0)
    vx = jnp.clip(ix_ref[...], 0, H - 1)
    vy = jnp.clip(iy_ref[...], 0, W - 1)
    seg_ref[...] = b_off + vx * W + vy


def _seg_call(ix, iy):
    nb = 32768
    return pl.pallas_call(
        _seg_body,
        grid=(BN // nb,),
        in_specs=[pl.BlockSpec((nb,), lambda i: (i,)),
                  pl.BlockSpec((nb,), lambda i: (i,))],
        out_specs=pl.BlockSpec((nb,), lambda i: (i,)),
        out_shape=jax.ShapeDtypeStruct((BN,), jnp.int32),
    )(ix, iy)


def _pro_body(fa_ref, fb_ref, fp_ref):
    fp_ref[:, 0:C] = fa_ref[0].T
    fp_ref[:, C:2 * C] = fb_ref[0].T


def _prologue(feat):
    nb = 8192
    nblk = N // nb
    return pl.pallas_call(
        _pro_body,
        grid=(nblk,),
        in_specs=[
            pl.BlockSpec((1, C, nb), lambda i: (0, 0, i)),
            pl.BlockSpec((1, C, nb), lambda i: (1, 0, i)),
        ],
        out_specs=pl.BlockSpec((nb, 2 * C), lambda i: (i, 0)),
        out_shape=jax.ShapeDtypeStruct((N, 2 * C), jnp.float32),
    )(feat, feat)


def _sc_body(feats_hbm, seg_hbm, temp_hbm,
             seg_buf, plist, cpid, clv, rows, accum, sem, sem_w):
    wid = lax.axis_index("s") * NC + lax.axis_index("c")
    lanes = lax.iota(jnp.int32, 16)

    def _pf(i, _):
        cpid[pl.ds(i * 16, 16)] = jnp.zeros((16,), jnp.int32)
        return 0
    lax.fori_loop(0, CLIST_CAP // 16, _pf, 0)

    pltpu.async_copy(seg_hbm.at[pl.ds(0, SEG_WIN)], seg_buf.at[0], sem)

    def _win(w, off):
        p = w & 1
        pltpu.make_async_copy(seg_hbm.at[pl.ds(w * SEG_WIN, SEG_WIN)],
                              seg_buf.at[p], sem).wait()

        @pl.when(w + 1 < NWIN)
        def _():
            pltpu.async_copy(
                seg_hbm.at[pl.ds((w + 1) * SEG_WIN, SEG_WIN)],
                seg_buf.at[(w + 1) & 1], sem)

        def _vreg8(q, off):
            svs, cnts = [], []
            for t in range(8):
                s = seg_buf[p, pl.ds(q * 128 + t * 16, 16)]
                own = ((s >> 10) & (NW - 1)) == wid
                cnts.append(plsc.all_reduce_population_count(own)[0])
                r = (s >> 15) & (ROUNDS - 1)
                lv = s & (CHUNK - 1)
                pid = w * SEG_WIN + q * 128 + t * 16 + lanes
                packed = (pid.astype(jnp.uint32) << 14) | \
                         ((r << 10) | lv).astype(jnp.uint32)
                key = jnp.where(own, jnp.uint32(0), jnp.uint32(1))
                _, sv = plsc.sort_key_val(key, packed)
                svs.append(sv)
            for t in range(8):
                o = jnp.minimum(off, LIST_CAP - 16)
                plist[pl.ds(o, 16)] = svs[t]
                off = off + cnts[t]
            return off
        return lax.fori_loop(0, SEG_WIN // 128, _vreg8, off)

    m_total = jnp.minimum(lax.fori_loop(0, NWIN, _win, jnp.int32(0)),
                          LIST_CAP)
    n_mv8 = (m_total + 127) // 128

    def _round(r, _):
        pair_base = (r * NW + wid) * (CHUNK // 2)

        def _scan4(q, k):
            svs, cnts = [], []
            for t in range(8):
                i0 = q * 128 + t * 16
                pk = plist[pl.ds(i0, 16)]
                valid = (i0 + lanes) < m_total
                r_of = ((pk >> 10) &
                        jnp.uint32(ROUNDS - 1)).astype(jnp.int32)
                sel = jnp.logical_and(r_of == r, valid)
                cnts.append(plsc.all_reduce_population_count(sel)[0])
                key = jnp.where(sel, jnp.uint32(0), jnp.uint32(1))
                _, sv = plsc.sort_key_val(key, pk)
                svs.append(sv)
            for t in range(8):
                o = jnp.minimum(k, CLIST_CAP - 16)
                cpid[pl.ds(o, 16)] = ((svs[t] >> 14) &
                                      jnp.uint32(N - 1)).astype(jnp.int32)
                clv[pl.ds(o, 16)] = ((svs[t] & jnp.uint32(CHUNK - 1)) |
                                     ((svs[t] >> 31) << 11)).astype(jnp.int32)
                k = k + cnts[t]
            return k
        k_total = jnp.minimum(lax.fori_loop(0, n_mv8, _scan4, jnp.int32(0)),
                              CLIST_CAP)
        clv[pl.ds(k_total, 16)] = jnp.full((16,), CHUNK, jnp.int32)

        @pl.when(r > 0)
        def _():
            pltpu.make_async_copy(
                accum.at[pl.ds(0, CHUNK // 2), :],
                temp_hbm.at[pl.ds(pair_base, CHUNK // 2), :], sem_w).wait()

        def _init(i, _):
            for u in range(4):
                for cg in range(8):
                    accum[i * 4 + u, pl.ds(cg * 16, 16)] = jnp.full(
                        (16,), _NEG_INF, jnp.float32)
            return 0
        lax.fori_loop(0, CHUNK // 2 // 4, _init, 0)

        nb = (k_total + GB - 1) // GB

        @pl.when(nb > 0)
        def _():
            pltpu.async_copy(
                feats_hbm.at[cpid.at[pl.ds(0, GB)]], rows.at[0], sem)

        def _batch(g, _):
            pb = g & 1
            pltpu.make_async_copy(
                feats_hbm.at[cpid.at[pl.ds(g * GB, GB)]],
                rows.at[pb], sem).wait()

            @pl.when(g + 1 < nb)
            def _():
                pltpu.async_copy(
                    feats_hbm.at[cpid.at[pl.ds((g + 1) * GB, GB)]],
                    rows.at[(g + 1) & 1], sem)
            cnt = jnp.minimum(k_total - g * GB, GB)

            def _grp(j, _):
                lvv = clv[pl.ds(g * GB + j * 16, 16)]
                for u in range(16):
                    e = lvv[u]
                    lv = e & 2047
                    hoff = ((e >> 11) & 1) << 6
                    trash = lv > CHUNK - 1
                    row = jnp.where(trash, CHUNK // 2, lv & (CHUNK // 2 - 1))
                    coff = jnp.where(trash, 0, (lv >> 9) << 6)
                    for cg in range(4):
                        a = accum[row, pl.ds(coff + cg * 16, 16)]
                        f = rows[pb, j * 16 + u, pl.ds(hoff + cg * 16, 16)]
                        accum[row, pl.ds(coff + cg * 16, 16)] = \
                            jnp.maximum(a, f)
                return 0
            return lax.fori_loop(0, (cnt + 15) // 16, _grp, 0)
        lax.fori_loop(0, nb, _batch, 0)

        pltpu.async_copy(accum.at[pl.ds(0, CHUNK // 2), :],
                         temp_hbm.at[pl.ds(pair_base, CHUNK // 2), :], sem_w)
        return 0
    lax.fori_loop(0, ROUNDS, _round, 0)
    pltpu.make_async_copy(
        accum.at[pl.ds(0, CHUNK // 2), :],
        temp_hbm.at[pl.ds((ROUNDS - 1) * NW * (CHUNK // 2), CHUNK // 2), :],
        sem_w).wait()


@functools.partial(
    pl.kernel,
    out_type=jax.ShapeDtypeStruct((BHW // 2, 2 * C), jnp.float32),
    mesh=plsc.VectorSubcoreMesh(core_axis_name="c", subcore_axis_name="s",
                                num_cores=NC, num_subcores=NS),
    scratch_types=[
        pltpu.VMEM((2, SEG_WIN), jnp.int32),
        pltpu.VMEM((LIST_CAP + 128,), jnp.uint32),
        pltpu.VMEM((CLIST_CAP,), jnp.int32),
        pltpu.VMEM((CLIST_CAP + 16,), jnp.int32),
        pltpu.VMEM((2, GB, 2 * C), jnp.float32),
        pltpu.VMEM((CHUNK // 2 + 1, 2 * C), jnp.float32),
        pltpu.SemaphoreType.DMA,
        pltpu.SemaphoreType.DMA,
    ],
    compiler_params=pltpu.CompilerParams(needs_layout_passes=False),
)
def _sc_kernel(feats_hbm, seg_hbm, temp_hbm,
               seg_buf, plist, cpid, clv, rows, accum, sem, sem_w):
    _sc_body(feats_hbm, seg_hbm, temp_hbm,
             seg_buf, plist, cpid, clv, rows, accum, sem, sem_w)


def _epi_body(temp_ref, out_ref):
    hc = CHUNK // 2
    for q in range(4):
        t = temp_ref[pl.ds(q * hc, hc), :]
        lo = t[:, 0:C]
        hi = t[:, C:2 * C]
        out_ref[0, :, 2 * q, :] = jnp.where(jnp.isfinite(lo), lo, 0.0).T
        out_ref[0, :, 2 * q + 1, :] = jnp.where(jnp.isfinite(hi), hi, 0.0).T


def _epilogue(temp):
    nck = 4
    per_b = NCHUNK // B // nck
    return pl.pallas_call(
        _epi_body,
        grid=(NCHUNK // nck,),
        in_specs=[pl.BlockSpec((nck * CHUNK // 2, 2 * C),
                               lambda i: (i, 0))],
        out_specs=pl.BlockSpec((1, C, 2 * nck, W),
                               lambda i: (i // per_b, 0, i % per_b, 0)),
        out_shape=jax.ShapeDtypeStruct((B, C, H, W), jnp.float32),
    )(temp)


def kernel(pcds_feat, pcds_ind):
    ix = pcds_ind[:, :, 0, 0].reshape(BN)
    iy = pcds_ind[:, :, 1, 0].reshape(BN)
    seg = _seg_call(ix, iy)
    feats_p = _prologue(pcds_feat[..., 0])
    temp = _sc_kernel(feats_p, seg)
    return _epilogue(temp)
